# Initial kernel scaffold; baseline (speedup 1.0000x reference)
import jax
import jax.numpy as jnp
from jax.experimental import pallas as pl


def kernel(x, edge_index, batch, W1, b1, W2, b2, W3, b3, Wm1, bm1, Wm2, bm2):
    raise NotImplementedError("write your pallas kernel here")


# speedup vs baseline: 11.8161x; 11.8161x over previous
import functools

import jax
import jax.numpy as jnp
from jax import lax
from jax.experimental import pallas as pl
from jax.experimental.pallas import tpu as pltpu
from jax.experimental.pallas import tpu_sc as plsc

N_ = 10000
D_ = 128
B_ = 64
NHID_ = 256
NOUT_ = 256
E_ = 320000

NPAD = 10240
EPAD = 331776
TPW = EPAD // 32
CH = 128
NIT = TPW // CH
ZR = NPAD // 16
DUMMY = NPAD - 1


@functools.lru_cache(maxsize=None)
def _get_deg_call():
    mesh = plsc.VectorSubcoreMesh(core_axis_name="c", subcore_axis_name="s")

    @functools.partial(
        pl.kernel,
        mesh=mesh,
        out_type=jax.ShapeDtypeStruct((32, NPAD), jnp.float32),
        scratch_types=[
            pltpu.VMEM((TPW // 8,), jnp.int32),
            pltpu.VMEM((NPAD,), jnp.float32),
        ],
        compiler_params=pltpu.CompilerParams(needs_layout_passes=False),
    )
    def deg_call(dst_hbm, zeros1_hbm, out_hbm, dstb, degv):
        cid = lax.axis_index("c")
        sid = lax.axis_index("s")
        wid = sid * 2 + cid
        pltpu.sync_copy(zeros1_hbm, degv)
        ones16 = jnp.ones((16,), jnp.float32)
        base = wid * TPW
        chunk = TPW // 8

        def body(i, carry):
            pltpu.sync_copy(dst_hbm.at[pl.ds(base + i * chunk, chunk)], dstb)
            for j in range(chunk // 16):
                idx = dstb[pl.ds(j * 16, 16)]
                plsc.addupdate_scatter(degv, [idx], ones16)
            return carry

        lax.fori_loop(0, 8, body, 0)
        pltpu.sync_copy(degv, out_hbm.at[wid])

    return deg_call


@functools.lru_cache(maxsize=None)
def _get_agg_call():
    mesh = plsc.VectorSubcoreMesh(core_axis_name="c", subcore_axis_name="s")

    @functools.partial(
        pl.kernel,
        mesh=mesh,
        out_type=jax.ShapeDtypeStruct((2, NPAD, 128), jnp.float32),
        scratch_types=[
            pltpu.VMEM((CH,), jnp.int32),
            pltpu.VMEM((CH,), jnp.int32),
            pltpu.VMEM((CH, 128), jnp.float32),
            pltpu.VMEM((128, 128), jnp.float32),
            pltpu.VMEM_SHARED((NPAD, 128), jnp.float32),
            pltpu.SemaphoreType.DMA,
        ],
        compiler_params=pltpu.CompilerParams(needs_layout_passes=False),
    )
    def agg_call(v_hbm, src_hbm, dst_hbm, zeros_hbm, out_hbm,
                 srcb, dstb, rows, stage, acc, sem):
        cid = lax.axis_index("c")
        sid = lax.axis_index("s")
        wid = sid * 2 + cid
        pltpu.sync_copy(zeros_hbm, stage)
        for z in range(ZR // 128):
            pltpu.sync_copy(stage, acc.at[pl.ds(sid * ZR + z * 128, 128)])
        plsc.subcore_barrier()

        base = wid * TPW

        def body(i, carry):
            off = base + i * CH
            pltpu.sync_copy(src_hbm.at[pl.ds(off, CH)], srcb)
            pltpu.async_copy(v_hbm.at[srcb], rows, sem).wait()
            pltpu.sync_copy(dst_hbm.at[pl.ds(off, CH)], dstb)
            pltpu.sync_copy(rows, acc.at[dstb], add=True)
            return carry

        lax.fori_loop(0, NIT, body, 0)
        plsc.subcore_barrier()
        for z in range(ZR // 128):
            r0 = sid * ZR + z * 128
            pltpu.sync_copy(acc.at[pl.ds(r0, 128)], stage)
            pltpu.sync_copy(stage, out_hbm.at[cid, pl.ds(r0, 128)])

    return agg_call


def _tc1_body(degT_ref, x_ref, w_ref, v_ref, dis_ref):
    deg = jnp.sum(degT_ref[...], axis=1, keepdims=True)
    dis = jnp.where(deg > 0, lax.rsqrt(jnp.maximum(deg, 1e-12)), 0.0)
    m = jnp.dot(x_ref[...], w_ref[...], preferred_element_type=jnp.float32)
    v_ref[...] = m * dis
    dis_ref[...] = dis


_tc1 = pl.pallas_call(
    _tc1_body,
    out_shape=[
        jax.ShapeDtypeStruct((NPAD, 128), jnp.float32),
        jax.ShapeDtypeStruct((NPAD, 1), jnp.float32),
    ],
)


def _tc2_body(agg_ref, dis_ref, b_ref, w_ref, v_ref):
    s = agg_ref[0] + agg_ref[1]
    h = jnp.maximum(s * dis_ref[...] + b_ref[...], 0.0)
    v_ref[...] = jnp.dot(h, w_ref[...],
                         preferred_element_type=jnp.float32) * dis_ref[...]


_tc2 = pl.pallas_call(
    _tc2_body,
    out_shape=jax.ShapeDtypeStruct((NPAD, 128), jnp.float32),
)


def _tc3_body(agg_ref, dis_ref, b3_ref, batch_ref,
              wm1_ref, bm1_ref, wm2_ref, bm2_ref, out_ref):
    h = (agg_ref[0] + agg_ref[1]) * dis_ref[...] + b3_ref[...]
    bids = batch_ref[...]
    sel = lax.broadcasted_iota(jnp.int32, (B_, NPAD), 0) == bids
    p = sel.astype(jnp.float32)
    sums = jnp.dot(p, h, preferred_element_type=jnp.float32)
    counts = jnp.sum(p, axis=1, keepdims=True)
    pooled = sums / jnp.maximum(counts, 1.0)
    z = jnp.maximum(
        jnp.dot(pooled, wm1_ref[...], preferred_element_type=jnp.float32)
        + bm1_ref[...], 0.0)
    out_ref[...] = jnp.dot(z, wm2_ref[...],
                           preferred_element_type=jnp.float32) + bm2_ref[...]


_tc3 = pl.pallas_call(
    _tc3_body,
    out_shape=jax.ShapeDtypeStruct((B_, NOUT_), jnp.float32),
)


def kernel(x, edge_index, batch, W1, b1, W2, b2, W3, b3, Wm1, bm1, Wm2, bm2):
    ei = edge_index.astype(jnp.int32)
    loop = jnp.arange(N_, dtype=jnp.int32)
    pad_e = EPAD - E_ - N_
    src = jnp.concatenate([ei[0], loop, jnp.zeros((pad_e,), jnp.int32)])
    dst = jnp.concatenate([ei[1], loop, jnp.full((pad_e,), DUMMY, jnp.int32)])
    xpad = jnp.zeros((NPAD, D_), jnp.float32).at[:N_].set(x)
    batch2d = jnp.concatenate(
        [batch.astype(jnp.int32), jnp.full((NPAD - N_,), B_, jnp.int32)]
    ).reshape(1, NPAD)
    zeros_blk = jnp.zeros((128, 128), jnp.float32)
    zeros1 = jnp.zeros((NPAD,), jnp.float32)

    deg_call = _get_deg_call()
    agg_call = _get_agg_call()
    deg_parts = deg_call(dst, zeros1)
    v1, dis = _tc1(deg_parts.T, xpad, W1)
    agg1 = agg_call(v1, src, dst, zeros_blk)
    v2 = _tc2(agg1, dis, b1.reshape(1, 128), W2)
    agg2 = agg_call(v2, src, dst, zeros_blk)
    v3 = _tc2(agg2, dis, b2.reshape(1, 128), W3)
    agg3 = agg_call(v3, src, dst, zeros_blk)
    out = _tc3(agg3, dis, b3.reshape(1, 128), batch2d,
               Wm1, bm1.reshape(1, NHID_), Wm2, bm2.reshape(1, NOUT_))
    return out
